# baseline (speedup 1.0000x reference)
import jax
import jax.numpy as jnp
from jax.experimental import pallas as pl
from jax.experimental.pallas import tpu as pltpu


def _gap_body(xa_ref, xb_ref, xc_ref, xd_ref, o_ref):
    inv = 1.0 / (4 * xa_ref.shape[0])
    s = (
        jnp.sum(xa_ref[...], axis=0, keepdims=True)
        + jnp.sum(xb_ref[...], axis=0, keepdims=True)
        + jnp.sum(xc_ref[...], axis=0, keepdims=True)
        + jnp.sum(xd_ref[...], axis=0, keepdims=True)
    )
    o_ref[0, ...] = s * inv


def kernel(x):
    b, c, h, w = x.shape
    hw = h * w
    quarter = hw // 4
    x2 = jnp.transpose(x, (0, 2, 3, 1)).reshape(b * hw, c)
    out = pl.pallas_call(
        _gap_body,
        out_shape=jax.ShapeDtypeStruct((b, 1, c), x.dtype),
        grid=(b,),
        in_specs=[
            pl.BlockSpec((quarter, c), lambda i: (4 * i, 0)),
            pl.BlockSpec((quarter, c), lambda i: (4 * i + 1, 0)),
            pl.BlockSpec((quarter, c), lambda i: (4 * i + 2, 0)),
            pl.BlockSpec((quarter, c), lambda i: (4 * i + 3, 0)),
        ],
        out_specs=pl.BlockSpec((1, 1, c), lambda i: (i, 0, 0)),
        compiler_params=pltpu.CompilerParams(
            dimension_semantics=("arbitrary",),
            vmem_limit_bytes=50 * 1024 * 1024,
        ),
    )(x2, x2, x2, x2)
    return out.reshape(b, c)


# speedup vs baseline: 1.0017x; 1.0017x over previous
import jax
import jax.numpy as jnp
from jax.experimental import pallas as pl
from jax.experimental.pallas import tpu as pltpu


def _gap_body(x_ref, o_ref):
    inv = 1.0 / x_ref.shape[0]
    o_ref[0, ...] = jnp.sum(x_ref[...], axis=0, keepdims=True) * inv


def kernel(x):
    b, c, h, w = x.shape
    hw = h * w
    x2 = jnp.transpose(x, (0, 2, 3, 1)).reshape(b * hw, c)
    out = pl.pallas_call(
        _gap_body,
        out_shape=jax.ShapeDtypeStruct((b, 1, c), x.dtype),
        grid=(b,),
        in_specs=[pl.BlockSpec((hw, c), lambda i: (i, 0))],
        out_specs=pl.BlockSpec((1, 1, c), lambda i: (i, 0, 0)),
        compiler_params=pltpu.CompilerParams(
            dimension_semantics=("arbitrary",),
            vmem_limit_bytes=50 * 1024 * 1024,
        ),
    )(x2)
    return out.reshape(b, c)
